# Initial kernel scaffold; baseline (speedup 1.0000x reference)
import jax
import jax.numpy as jnp
from jax.experimental import pallas as pl


def kernel(pred, target, trans_feat, feature_dense, feature_sparse):
    raise NotImplementedError("write your pallas kernel here")


# speedup vs baseline: 1.3755x; 1.3755x over previous
import jax
import jax.numpy as jnp
from jax import lax
from jax.experimental import pallas as pl
from jax.experimental.pallas import tpu as pltpu

_INTERPRET = False

_B = 4096
_N = 8192
_D = 1024
_NC = 40
_K = 64

_BB = 256
_NS = _B // _BB // 2

_BM = 512
_NB = _N // _BM
_NK = _NB // 2 + 1
_GI = (_NB * _NK) // 2

_LOG2E = 1.4426950408889634


def _stats_kernel(pred_ref, tgt_ref, tf_ref, tf2_ref, d_ref, s_ref,
                  sqd_ref, sqs_ref, u_ref, scal_ref):
    s_idx = pl.program_id(1)

    p = pred_ref[0]
    tgt = tgt_ref[...]
    hit = lax.broadcasted_iota(jnp.int32, (_BB, _NC), 1) == tgt
    nll_part = jnp.sum(jnp.where(hit, p, 0.0))

    t = tf_ref[...]
    ttt = lax.dot_general(t, t, (((2,), (2,)), ((0,), (0,))),
                          preferred_element_type=jnp.float32)
    g2 = jnp.sum(jnp.sum(ttt * ttt, axis=1), axis=1, keepdims=True)
    t2 = tf2_ref[...]
    tn = jnp.sum(t2 * t2, axis=1, keepdims=True)
    ssq = g2 - 2.0 * tn + float(_K)
    md_part = jnp.sum(jnp.sqrt(jnp.maximum(ssq, 0.0)))

    d = d_ref[...]
    sp = s_ref[...]
    sqd = jnp.sum(d * d, axis=1, keepdims=True)
    sqs = jnp.sum(sp * sp, axis=1, keepdims=True)
    sqd_ref[...] = sqd
    sqs_ref[...] = sqs
    s_part = jnp.sum(sqd) + jnp.sum(sqs)
    u_part = (jnp.sum(d, axis=0, keepdims=True)
              + jnp.sum(sp, axis=0, keepdims=True))

    @pl.when(s_idx == 0)
    def _():
        u_ref[...] = jnp.zeros_like(u_ref)
        scal_ref[...] = jnp.zeros_like(scal_ref)

    u_ref[0] += u_part
    lane = lax.broadcasted_iota(jnp.int32, (1, 1, 128), 2)
    scal_ref[...] += jnp.where(
        lane == 0, nll_part,
        jnp.where(lane == 1, md_part,
                  jnp.where(lane == 2, s_part, 0.0)))


def _mmd_kernel(a_ref, b_ref, sqc_ref, sqr_ref, acc_ref):
    c = pl.program_id(0)
    s = pl.program_id(1)
    t = c * _GI + s
    i = t // _NK
    k = t % _NK
    j = (i + k) % _NB

    a = a_ref[...]
    b = b_ref[...]
    g = lax.dot_general(a, b, (((0,), (0,)), ((), ())),
                        preferred_element_type=jnp.float32)
    l2s = jnp.minimum(sqc_ref[...] + sqr_ref[...] + g, 0.0)
    e = jnp.exp2(l2s)
    ks = e
    p2 = e * e
    ks += p2
    p4 = p2 * p2
    ks += p4
    p8 = p4 * p4
    ks += p8
    ks += p8 * p8
    bs = jnp.sum(ks)

    w = jnp.where((k >= 1) & (k < _NB // 2), 2.0, 1.0)
    half = _NB // 2
    i_ge = i >= half
    j_ge = j >= half
    region = jnp.where(i_ge == j_ge, jnp.where(i_ge, 1, 0), 2)
    lane = lax.broadcasted_iota(jnp.int32, (1, 1, 128), 2)

    @pl.when(s == 0)
    def _():
        acc_ref[...] = jnp.zeros_like(acc_ref)

    acc_ref[...] += jnp.where(lane == region, bs * w, 0.0)


def kernel(pred, target, trans_feat, feature_dense, feature_sparse):
    pred3 = pred.reshape(_B // _BB, _BB, _NC)
    tgt2 = target.astype(jnp.int32).reshape(_B, 1)
    tf2 = trans_feat.reshape(_B, _K * _K)

    sq_d, sq_s, u_out, scal = pl.pallas_call(
        _stats_kernel,
        grid=(2, _NS),
        in_specs=[
            pl.BlockSpec((1, _BB, _NC), lambda c, s: (c * _NS + s, 0, 0)),
            pl.BlockSpec((_BB, 1), lambda c, s: (c * _NS + s, 0)),
            pl.BlockSpec((_BB, _K, _K), lambda c, s: (c * _NS + s, 0, 0)),
            pl.BlockSpec((_BB, _K * _K), lambda c, s: (c * _NS + s, 0)),
            pl.BlockSpec((_BB, _D), lambda c, s: (c * _NS + s, 0)),
            pl.BlockSpec((_BB, _D), lambda c, s: (c * _NS + s, 0)),
        ],
        out_specs=[
            pl.BlockSpec((_BB, 1), lambda c, s: (c * _NS + s, 0)),
            pl.BlockSpec((_BB, 1), lambda c, s: (c * _NS + s, 0)),
            pl.BlockSpec((1, 1, _D), lambda c, s: (c, 0, 0)),
            pl.BlockSpec((1, 1, 128), lambda c, s: (c, 0, 0)),
        ],
        out_shape=[
            jax.ShapeDtypeStruct((_B, 1), jnp.float32),
            jax.ShapeDtypeStruct((_B, 1), jnp.float32),
            jax.ShapeDtypeStruct((2, 1, _D), jnp.float32),
            jax.ShapeDtypeStruct((2, 1, 128), jnp.float32),
        ],
        compiler_params=pltpu.CompilerParams(
            dimension_semantics=("parallel", "arbitrary"),
        ),
        name="mmd_stats",
        interpret=_INTERPRET,
    )(pred3, tgt2, trans_feat, tf2, feature_dense, feature_sparse)

    scal_sum = jnp.sum(scal, axis=(0, 1))
    nll = -scal_sum[0] / _B
    md = scal_sum[1] / _B
    s_tot = scal_sum[2]
    u = jnp.sum(u_out, axis=(0, 1))
    uu = jnp.vdot(u, u)
    sum_l2 = 2.0 * _N * s_tot - 2.0 * uu
    bandwidth = sum_l2 / (float(_N) ** 2 - float(_N))
    c2 = (0.25 * _LOG2E) / bandwidth

    scale = jnp.sqrt(2.0 * c2)
    total_t = jnp.concatenate([feature_dense, feature_sparse], axis=0).T
    total_ts = (total_t * scale).astype(jnp.bfloat16)
    sq_col = jnp.concatenate([sq_d, sq_s], axis=0) * (-c2)
    sq_row = sq_col.reshape(1, _N)

    acc = pl.pallas_call(
        _mmd_kernel,
        grid=(2, _GI),
        in_specs=[
            pl.BlockSpec((_D, _BM),
                         lambda c, s: (0, (c * _GI + s) // _NK)),
            pl.BlockSpec((_D, _BM),
                         lambda c, s: (0, ((c * _GI + s) // _NK
                                           + (c * _GI + s) % _NK) % _NB)),
            pl.BlockSpec((_BM, 1),
                         lambda c, s: ((c * _GI + s) // _NK, 0)),
            pl.BlockSpec((1, _BM),
                         lambda c, s: (0, ((c * _GI + s) // _NK
                                           + (c * _GI + s) % _NK) % _NB)),
        ],
        out_specs=pl.BlockSpec((1, 1, 128), lambda c, s: (c, 0, 0)),
        out_shape=jax.ShapeDtypeStruct((2, 1, 128), jnp.float32),
        compiler_params=pltpu.CompilerParams(
            dimension_semantics=("parallel", "arbitrary"),
        ),
        name="mmd_gram",
        interpret=_INTERPRET,
    )(total_ts, total_ts, sq_col, sq_row)

    accs = jnp.sum(acc, axis=(0, 1))
    mmd = (accs[0] + accs[1] - accs[2]) / (float(_B) * float(_B))

    return 0.1 * nll + 0.001 * md + 0.5 * mmd


# speedup vs baseline: 1.5855x; 1.1527x over previous
import functools

import jax
import jax.numpy as jnp
from jax import lax
from jax.experimental import pallas as pl
from jax.experimental.pallas import tpu as pltpu

_INTERPRET = False

_B = 4096
_N = 8192
_D = 1024
_NC = 40
_K = 64

_BB = 256
_NS = _B // _BB

_BM = 1024
_CH = 256
_NB = _B // _BM
_NK = _NB // 2 + 1

_LOG2E = 1.4426950408889634


def _stats_kernel(pred_ref, tgt_ref, tf_ref, tf2_ref, d_ref, s_ref,
                  sqd_ref, sqs_ref, u_ref, scal_ref):
    step = pl.program_id(0)

    p = pred_ref[0]
    tgt = tgt_ref[...]
    hit = lax.broadcasted_iota(jnp.int32, (_BB, _NC), 1) == tgt
    nll_part = jnp.sum(jnp.where(hit, p, 0.0))

    t = tf_ref[...]
    ttt = lax.dot_general(t, t, (((2,), (2,)), ((0,), (0,))),
                          preferred_element_type=jnp.float32)
    g2 = jnp.sum(jnp.sum(ttt * ttt, axis=1), axis=1, keepdims=True)
    t2 = tf2_ref[...]
    tn = jnp.sum(t2 * t2, axis=1, keepdims=True)
    ssq = g2 - 2.0 * tn + float(_K)
    md_part = jnp.sum(jnp.sqrt(jnp.maximum(ssq, 0.0)))

    d = d_ref[...]
    sp = s_ref[...]
    sqd = jnp.sum(d * d, axis=1, keepdims=True)
    sqs = jnp.sum(sp * sp, axis=1, keepdims=True)
    sqd_ref[...] = sqd
    sqs_ref[...] = sqs
    s_part = jnp.sum(sqd) + jnp.sum(sqs)
    u_part = (jnp.sum(d, axis=0, keepdims=True)
              + jnp.sum(sp, axis=0, keepdims=True))

    @pl.when(step == 0)
    def _():
        u_ref[...] = jnp.zeros_like(u_ref)
        scal_ref[...] = jnp.zeros_like(scal_ref)

    u_ref[0] += u_part
    lane = lax.broadcasted_iota(jnp.int32, (1, 1, 128), 2)
    scal_ref[...] += jnp.where(
        lane == 0, nll_part,
        jnp.where(lane == 1, md_part,
                  jnp.where(lane == 2, s_part, 0.0)))


def _gram_kernel(a_ref, b_ref, sqc_ref, sqr_ref, acc_ref, *, sym):
    t = pl.program_id(0)
    if sym:
        k = t % _NK
        w = jnp.where((k >= 1) & (k < _NB // 2), 2.0, 1.0)
    else:
        w = 1.0

    parts = []
    for ic in range(_BM // _CH):
        a_c = a_ref[ic * _CH:(ic + 1) * _CH, :]
        sqc_c = sqc_ref[ic * _CH:(ic + 1) * _CH, :]
        for jc in range(_BM // _CH):
            b_c = b_ref[jc * _CH:(jc + 1) * _CH, :]
            sqr_c = sqr_ref[:, jc * _CH:(jc + 1) * _CH]
            g = lax.dot_general(a_c, b_c, (((1,), (1,)), ((), ())),
                                preferred_element_type=jnp.float32)
            l2s = jnp.minimum(sqc_c + sqr_c + g, 0.0)
            e = jnp.exp2(l2s)
            ks = e
            p2 = e * e
            ks += p2
            p4 = p2 * p2
            ks += p4
            p8 = p4 * p4
            ks += p8
            ks += p8 * p8
            parts.append(jnp.sum(ks, axis=0, keepdims=True))
    bs = jnp.sum(sum(parts)) * w

    lane = lax.broadcasted_iota(jnp.int32, (1, 1, 128), 2)

    @pl.when(t == 0)
    def _():
        acc_ref[...] = jnp.zeros_like(acc_ref)

    acc_ref[...] += jnp.where(lane == 0, bs, 0.0)


def _gram_call(feat_a, feat_b, sqc_a, sqr_b, *, sym, name):
    if sym:
        grid = (_NB * _NK,)
        i_map = lambda t: t // _NK
        j_map = lambda t: (t // _NK + t % _NK) % _NB
    else:
        grid = (_NB * _NB,)
        i_map = lambda t: t // _NB
        j_map = lambda t: t % _NB
    acc = pl.pallas_call(
        functools.partial(_gram_kernel, sym=sym),
        grid=grid,
        in_specs=[
            pl.BlockSpec((_BM, _D), lambda t: (i_map(t), 0)),
            pl.BlockSpec((_BM, _D), lambda t: (j_map(t), 0)),
            pl.BlockSpec((_BM, 1), lambda t: (i_map(t), 0)),
            pl.BlockSpec((1, _BM), lambda t: (0, j_map(t))),
        ],
        out_specs=pl.BlockSpec((1, 1, 128), lambda t: (0, 0, 0)),
        out_shape=jax.ShapeDtypeStruct((1, 1, 128), jnp.float32),
        compiler_params=pltpu.CompilerParams(
            dimension_semantics=("arbitrary",),
        ),
        name=name,
        interpret=_INTERPRET,
    )(feat_a, feat_b, sqc_a, sqr_b)
    return acc[0, 0, 0]


def kernel(pred, target, trans_feat, feature_dense, feature_sparse):
    pred3 = pred.reshape(_B // _BB, _BB, _NC)
    tgt2 = target.astype(jnp.int32).reshape(_B, 1)
    tf2 = trans_feat.reshape(_B, _K * _K)

    sq_d, sq_s, u_out, scal = pl.pallas_call(
        _stats_kernel,
        grid=(_NS,),
        in_specs=[
            pl.BlockSpec((1, _BB, _NC), lambda s: (s, 0, 0)),
            pl.BlockSpec((_BB, 1), lambda s: (s, 0)),
            pl.BlockSpec((_BB, _K, _K), lambda s: (s, 0, 0)),
            pl.BlockSpec((_BB, _K * _K), lambda s: (s, 0)),
            pl.BlockSpec((_BB, _D), lambda s: (s, 0)),
            pl.BlockSpec((_BB, _D), lambda s: (s, 0)),
        ],
        out_specs=[
            pl.BlockSpec((_BB, 1), lambda s: (s, 0)),
            pl.BlockSpec((_BB, 1), lambda s: (s, 0)),
            pl.BlockSpec((1, 1, _D), lambda s: (0, 0, 0)),
            pl.BlockSpec((1, 1, 128), lambda s: (0, 0, 0)),
        ],
        out_shape=[
            jax.ShapeDtypeStruct((_B, 1), jnp.float32),
            jax.ShapeDtypeStruct((_B, 1), jnp.float32),
            jax.ShapeDtypeStruct((1, 1, _D), jnp.float32),
            jax.ShapeDtypeStruct((1, 1, 128), jnp.float32),
        ],
        compiler_params=pltpu.CompilerParams(
            dimension_semantics=("arbitrary",),
        ),
        name="mmd_stats",
        interpret=_INTERPRET,
    )(pred3, tgt2, trans_feat, tf2, feature_dense, feature_sparse)

    nll = -scal[0, 0, 0] / _B
    md = scal[0, 0, 1] / _B
    s_tot = scal[0, 0, 2]
    u = u_out.reshape(_D)
    uu = jnp.vdot(u, u)
    sum_l2 = 2.0 * _N * s_tot - 2.0 * uu
    bandwidth = sum_l2 / (float(_N) ** 2 - float(_N))
    c2 = (0.25 * _LOG2E) / bandwidth

    scale = jnp.sqrt(2.0 * c2).astype(jnp.float32)
    dense_sb = (feature_dense * scale).astype(jnp.bfloat16)
    sparse_sb = (feature_sparse * scale).astype(jnp.bfloat16)
    sqc_d = sq_d * (-c2)
    sqc_s = sq_s * (-c2)
    sqr_d = sqc_d.reshape(1, _B)
    sqr_s = sqc_s.reshape(1, _B)

    sxx = _gram_call(dense_sb, dense_sb, sqc_d, sqr_d, sym=True, name="mmd_xx")
    syy = _gram_call(sparse_sb, sparse_sb, sqc_s, sqr_s, sym=True,
                     name="mmd_yy")
    sxy = _gram_call(dense_sb, sparse_sb, sqc_d, sqr_s, sym=False,
                     name="mmd_xy")

    mmd = (sxx + syy - 2.0 * sxy) / (float(_B) * float(_B))

    return 0.1 * nll + 0.001 * md + 0.5 * mmd


# speedup vs baseline: 1.8542x; 1.1695x over previous
import functools

import jax
import jax.numpy as jnp
from jax import lax
from jax.experimental import pallas as pl
from jax.experimental.pallas import tpu as pltpu

_INTERPRET = False

_B = 4096
_N = 8192
_D = 1024
_NC = 40
_K = 64

_BB = 256
_NS = _B // _BB

_BM = 1024
_CH = 256
_NB = _B // _BM
_NK = _NB // 2 + 1

_LOG2E = 1.4426950408889634


def _stats_kernel(pred_ref, tgt_ref, tf_ref, d_ref, s_ref,
                  sqd_ref, sqs_ref, u_ref, scal_ref):
    step = pl.program_id(0)

    p = pred_ref[...]
    tgt = tgt_ref[...]
    hit = lax.broadcasted_iota(jnp.int32, (_BB, _NC), 1) == tgt
    nll_part = jnp.sum(jnp.where(hit, p, 0.0))

    t = tf_ref[...]
    ttt = lax.dot_general(t, t, (((2,), (2,)), ((0,), (0,))),
                          preferred_element_type=jnp.float32)
    g2 = jnp.sum(jnp.sum(ttt * ttt, axis=1), axis=1, keepdims=True)
    tn = jnp.sum(jnp.sum(t * t, axis=1), axis=1, keepdims=True)
    ssq = g2 - 2.0 * tn + float(_K)
    md_part = jnp.sum(jnp.sqrt(jnp.maximum(ssq, 0.0)))

    d = d_ref[...]
    sp = s_ref[...]
    sqd = jnp.sum(d * d, axis=1, keepdims=True)
    sqs = jnp.sum(sp * sp, axis=1, keepdims=True)
    sqd_ref[...] = sqd
    sqs_ref[...] = sqs
    s_part = jnp.sum(sqd) + jnp.sum(sqs)
    u_part = (jnp.sum(d, axis=0, keepdims=True)
              + jnp.sum(sp, axis=0, keepdims=True))

    @pl.when(step == 0)
    def _():
        u_ref[...] = jnp.zeros_like(u_ref)
        scal_ref[...] = jnp.zeros_like(scal_ref)

    u_ref[0] += u_part
    lane = lax.broadcasted_iota(jnp.int32, (1, 1, 128), 2)
    scal_ref[...] += jnp.where(
        lane == 0, nll_part,
        jnp.where(lane == 1, md_part,
                  jnp.where(lane == 2, s_part, 0.0)))


def _gram_kernel(a_ref, b_ref, sqc_ref, sqr_ref, acc_ref, *, sym):
    t = pl.program_id(0)
    if sym:
        k = t % _NK
        w = jnp.where((k >= 1) & (k < _NB // 2), 2.0, 1.0)
    else:
        w = 1.0

    parts = []
    for ic in range(_BM // _CH):
        a_c = a_ref[ic * _CH:(ic + 1) * _CH, :]
        sqc_c = sqc_ref[ic * _CH:(ic + 1) * _CH, :]
        for jc in range(_BM // _CH):
            b_c = b_ref[jc * _CH:(jc + 1) * _CH, :]
            sqr_c = sqr_ref[:, jc * _CH:(jc + 1) * _CH]
            g = lax.dot_general(a_c, b_c, (((1,), (1,)), ((), ())),
                                preferred_element_type=jnp.float32)
            l2s = jnp.minimum(sqc_c + sqr_c + g, 0.0)
            e = jnp.exp2(l2s)
            ks = e
            p2 = e * e
            ks += p2
            p4 = p2 * p2
            ks += p4
            p8 = p4 * p4
            ks += p8
            ks += p8 * p8
            parts.append(jnp.sum(ks, axis=0, keepdims=True))
    bs = jnp.sum(sum(parts)) * w

    lane = lax.broadcasted_iota(jnp.int32, (1, 1, 128), 2)

    @pl.when(t == 0)
    def _():
        acc_ref[...] = jnp.zeros_like(acc_ref)

    acc_ref[...] += jnp.where(lane == 0, bs, 0.0)


def _gram_call(feat_a, feat_b, sqc_a, sqr_b, *, sym, name):
    if sym:
        grid = (_NB * _NK,)
        i_map = lambda t: t // _NK
        j_map = lambda t: (t // _NK + t % _NK) % _NB
    else:
        grid = (_NB * _NB,)
        i_map = lambda t: t // _NB
        j_map = lambda t: t % _NB
    acc = pl.pallas_call(
        functools.partial(_gram_kernel, sym=sym),
        grid=grid,
        in_specs=[
            pl.BlockSpec((_BM, _D), lambda t: (i_map(t), 0)),
            pl.BlockSpec((_BM, _D), lambda t: (j_map(t), 0)),
            pl.BlockSpec((_BM, 1), lambda t: (i_map(t), 0)),
            pl.BlockSpec((1, _BM), lambda t: (0, j_map(t))),
        ],
        out_specs=pl.BlockSpec((1, 1, 128), lambda t: (0, 0, 0)),
        out_shape=jax.ShapeDtypeStruct((1, 1, 128), jnp.float32),
        compiler_params=pltpu.CompilerParams(
            dimension_semantics=("arbitrary",),
        ),
        name=name,
        interpret=_INTERPRET,
    )(feat_a, feat_b, sqc_a, sqr_b)
    return acc[0, 0, 0]


def kernel(pred, target, trans_feat, feature_dense, feature_sparse):
    tgt2 = target.astype(jnp.int32).reshape(_B, 1)

    sq_d, sq_s, u_out, scal = pl.pallas_call(
        _stats_kernel,
        grid=(_NS,),
        in_specs=[
            pl.BlockSpec((_BB, _NC), lambda s: (s, 0)),
            pl.BlockSpec((_BB, 1), lambda s: (s, 0)),
            pl.BlockSpec((_BB, _K, _K), lambda s: (s, 0, 0)),
            pl.BlockSpec((_BB, _D), lambda s: (s, 0)),
            pl.BlockSpec((_BB, _D), lambda s: (s, 0)),
        ],
        out_specs=[
            pl.BlockSpec((_BB, 1), lambda s: (s, 0)),
            pl.BlockSpec((_BB, 1), lambda s: (s, 0)),
            pl.BlockSpec((1, 1, _D), lambda s: (0, 0, 0)),
            pl.BlockSpec((1, 1, 128), lambda s: (0, 0, 0)),
        ],
        out_shape=[
            jax.ShapeDtypeStruct((_B, 1), jnp.float32),
            jax.ShapeDtypeStruct((_B, 1), jnp.float32),
            jax.ShapeDtypeStruct((1, 1, _D), jnp.float32),
            jax.ShapeDtypeStruct((1, 1, 128), jnp.float32),
        ],
        compiler_params=pltpu.CompilerParams(
            dimension_semantics=("arbitrary",),
        ),
        name="mmd_stats",
        interpret=_INTERPRET,
    )(pred, tgt2, trans_feat, feature_dense, feature_sparse)

    nll = -scal[0, 0, 0] / _B
    md = scal[0, 0, 1] / _B
    s_tot = scal[0, 0, 2]
    u = u_out.reshape(_D)
    uu = jnp.vdot(u, u)
    sum_l2 = 2.0 * _N * s_tot - 2.0 * uu
    bandwidth = sum_l2 / (float(_N) ** 2 - float(_N))
    c2 = (0.25 * _LOG2E) / bandwidth

    scale = jnp.sqrt(2.0 * c2).astype(jnp.float32)
    dense_sb = (feature_dense * scale).astype(jnp.bfloat16)
    sparse_sb = (feature_sparse * scale).astype(jnp.bfloat16)
    sqc_d = sq_d * (-c2)
    sqc_s = sq_s * (-c2)
    sqr_d = sqc_d.reshape(1, _B)
    sqr_s = sqc_s.reshape(1, _B)

    sxx = _gram_call(dense_sb, dense_sb, sqc_d, sqr_d, sym=True, name="mmd_xx")
    syy = _gram_call(sparse_sb, sparse_sb, sqc_s, sqr_s, sym=True,
                     name="mmd_yy")
    sxy = _gram_call(dense_sb, sparse_sb, sqc_d, sqr_s, sym=False,
                     name="mmd_xy")

    mmd = (sxx + syy - 2.0 * sxy) / (float(_B) * float(_B))

    return 0.1 * nll + 0.001 * md + 0.5 * mmd
